# Initial kernel scaffold; baseline (speedup 1.0000x reference)
import functools

import jax
import jax.numpy as jnp
from jax import lax
from jax.experimental import pallas as pl
from jax.experimental.pallas import tpu as pltpu
from jax.experimental.pallas import tpu_sc as plsc

N = 10000
E = 320000
D = 128
ED = 16

NP = 10240
DP = 144
K = 128
NCHUNKS = E // K
NCORES = 2
NSUB = 16
NW = NCORES * NSUB
ROWS_PER_SUB = NP // NSUB
CPB = 128
NCOPY = ROWS_PER_SUB // CPB


_EB = 6400


def _edge_w_body(ea_ref, w12_ref, b12_ref, w1_ref, w2_ref):
    z = jnp.dot(ea_ref[...], w12_ref[...], preferred_element_type=jnp.float32)
    z = z + b12_ref[...]
    w = jnp.maximum(z, 0.0) + jnp.log(1.0 + jnp.exp(-jnp.abs(z)))
    w1_ref[...] = w[:, 0:1]
    w2_ref[...] = w[:, 1:2]


def _edge_weights(edge_attr, W12, b12):
    return pl.pallas_call(
        _edge_w_body,
        grid=(E // _EB,),
        in_specs=[
            pl.BlockSpec((_EB, ED), lambda i: (i, 0)),
            pl.BlockSpec((ED, 2), lambda i: (0, 0)),
            pl.BlockSpec((1, 2), lambda i: (0, 0)),
        ],
        out_specs=[
            pl.BlockSpec((_EB, 1), lambda i: (i, 0)),
            pl.BlockSpec((_EB, 1), lambda i: (i, 0)),
        ],
        out_shape=[
            jax.ShapeDtypeStruct((E, 1), jnp.float32),
            jax.ShapeDtypeStruct((E, 1), jnp.float32),
        ],
    )(edge_attr, W12, b12)


def _spmm_body(hp_hbm, src_hbm, dst_hbm, w_hbm, out_hbm,
               sidx_v, didx_v, w_v, rows_v, buf_v, acc_sh, sem):
    cid = lax.axis_index("c")
    sid = lax.axis_index("s")
    wid = sid * NCORES + cid

    def _zb(r, carry):
        for c in range(DP // 16):
            buf_v[r, pl.ds(c * 16, 16)] = jnp.zeros((16,), jnp.float32)
        return carry

    lax.fori_loop(0, CPB, _zb, 0)
    for b in range(NCOPY):
        pltpu.sync_copy(buf_v, acc_sh.at[pl.ds(sid * ROWS_PER_SUB + b * CPB, CPB)])
    plsc.subcore_barrier()

    base_chunks = NCHUNKS // NW
    extra = NCHUNKS - base_chunks * NW
    my_n = base_chunks + jnp.where(wid < extra, 1, 0)

    def _chunk(t, carry):
        base = (wid + t * NW) * K
        pltpu.sync_copy(src_hbm.at[pl.ds(base, K)], sidx_v)
        pltpu.sync_copy(dst_hbm.at[pl.ds(base, K)], didx_v)
        pltpu.sync_copy(w_hbm.at[pl.ds(base, K)], w_v)
        pltpu.async_copy(hp_hbm.at[sidx_v], rows_v, sem).wait()

        def _scale(j, c2):
            wspl = plsc.load_gather(w_v, [jnp.full((16,), j, jnp.int32)])
            for c in range(DP // 16):
                rows_v[j, pl.ds(c * 16, 16)] = rows_v[j, pl.ds(c * 16, 16)] * wspl
            return c2

        lax.fori_loop(0, K, _scale, 0)
        pltpu.sync_copy(rows_v, acc_sh.at[didx_v], add=True)
        return carry

    lax.fori_loop(0, my_n, _chunk, 0)
    plsc.subcore_barrier()

    for b in range(NCOPY):
        r0 = sid * ROWS_PER_SUB + b * CPB
        pltpu.sync_copy(acc_sh.at[pl.ds(r0, CPB)], buf_v)
        pltpu.sync_copy(buf_v, out_hbm.at[cid, pl.ds(r0, CPB)])


_spmm = pl.kernel(
    _spmm_body,
    out_type=jax.ShapeDtypeStruct((NCORES, NP, DP), jnp.float32),
    mesh=plsc.VectorSubcoreMesh(core_axis_name="c", subcore_axis_name="s"),
    scratch_types=[
        pltpu.VMEM((K,), jnp.int32),
        pltpu.VMEM((K,), jnp.int32),
        pltpu.VMEM((K,), jnp.float32),
        pltpu.VMEM((K, DP), jnp.float32),
        pltpu.VMEM((CPB, DP), jnp.float32),
        pltpu.VMEM_SHARED((NP, DP), jnp.float32),
        pltpu.SemaphoreType.DMA,
    ],
)


_RB = 512


def _layer_body(final, p_ref, hp_ref, wl_ref, bl_ref, g_ref, be_ref,
                wo_ref, bo_ref, out_ref):
    s = p_ref[0] + p_ref[1]
    c = s[:, D:D + 1]
    h = hp_ref[:, :D]
    agg = s[:, :D] - c * h
    z = jnp.dot(agg, wl_ref[...], preferred_element_type=jnp.float32) + bl_ref[...]
    r = jnp.maximum(z, 0.0)
    mu = jnp.mean(r, axis=-1, keepdims=True)
    dev = r - mu
    var = jnp.mean(dev * dev, axis=-1, keepdims=True)
    ln = dev * lax.rsqrt(var + 1e-5) * g_ref[...] + be_ref[...]
    h2 = ln + h
    if final:
        out_ref[...] = (
            jnp.dot(h2, wo_ref[...], preferred_element_type=jnp.float32) + bo_ref[...]
        )
    else:
        out_ref[:, :D] = h2
        out_ref[:, D:D + 1] = jnp.ones((_RB, 1), jnp.float32)
        out_ref[:, D + 1:] = jnp.zeros((_RB, DP - D - 1), jnp.float32)


def _layer_tc(final, P, hp, W_l, b_l, g, be, W_o, b_o):
    odim = D if final else DP
    return pl.pallas_call(
        functools.partial(_layer_body, final),
        grid=(NP // _RB,),
        in_specs=[
            pl.BlockSpec((NCORES, _RB, DP), lambda i: (0, i, 0)),
            pl.BlockSpec((_RB, DP), lambda i: (i, 0)),
            pl.BlockSpec((D, D), lambda i: (0, 0)),
            pl.BlockSpec((1, D), lambda i: (0, 0)),
            pl.BlockSpec((1, D), lambda i: (0, 0)),
            pl.BlockSpec((1, D), lambda i: (0, 0)),
            pl.BlockSpec((D, D), lambda i: (0, 0)),
            pl.BlockSpec((1, D), lambda i: (0, 0)),
        ],
        out_specs=pl.BlockSpec((_RB, odim), lambda i: (i, 0)),
        out_shape=jax.ShapeDtypeStruct((NP, odim), jnp.float32),
    )(P, hp, W_l, b_l, g, be, W_o, b_o)


def kernel(x, edge_index, edge_attr, W_e1, b_e1, W_l1, b_l1, g1, be1,
           W_e2, b_e2, W_l2, b_l2, g2, be2, W_out, b_out):
    src = edge_index[0]
    dst = edge_index[1]

    W12 = jnp.concatenate([W_e1, W_e2], axis=1)
    b12 = jnp.stack([b_e1[0], b_e2[0]]).reshape(1, 2)
    w1, w2 = _edge_weights(edge_attr, W12, b12)
    w1 = w1.reshape(E)
    w2 = w2.reshape(E)

    xp = jnp.zeros((NP, DP), jnp.float32)
    xp = xp.at[:N, :D].set(x)
    xp = xp.at[:, D].set(1.0)

    b_l1r = b_l1.reshape(1, D)
    g1r = g1.reshape(1, D)
    be1r = be1.reshape(1, D)
    b_l2r = b_l2.reshape(1, D)
    g2r = g2.reshape(1, D)
    be2r = be2.reshape(1, D)
    b_or = b_out.reshape(1, D)

    P1 = _spmm(xp, src, dst, w1)
    h1p = _layer_tc(False, P1, xp, W_l1, b_l1r, g1r, be1r, W_out, b_or)
    P2 = _spmm(h1p, src, dst, w2)
    out = _layer_tc(True, P2, h1p, W_l2, b_l2r, g2r, be2r, W_out, b_or)
    return out[:N]


# speedup vs baseline: 4.0466x; 4.0466x over previous
import functools

import jax
import jax.numpy as jnp
from jax import lax
from jax.experimental import pallas as pl
from jax.experimental.pallas import tpu as pltpu
from jax.experimental.pallas import tpu_sc as plsc

N = 10000
E = 320000
D = 128
ED = 16

NP = 10240
DP = 144
K = 128
NCHUNKS = E // K
NCORES = 2
NSUB = 16
NW = NCORES * NSUB
ROWS_PER_SUB = NP // NSUB
CPB = 128
NCOPY = ROWS_PER_SUB // CPB


_EB = 6400


def _edge_w_body(ea_ref, w12_ref, b12_ref, w1_ref, w2_ref):
    z = jnp.dot(ea_ref[...], w12_ref[...], preferred_element_type=jnp.float32)
    z = z + b12_ref[...]
    w = jnp.maximum(z, 0.0) + jnp.log(1.0 + jnp.exp(-jnp.abs(z)))
    w1_ref[...] = w[:, 0:1]
    w2_ref[...] = w[:, 1:2]


def _edge_weights(edge_attr, W12, b12):
    return pl.pallas_call(
        _edge_w_body,
        grid=(E // _EB,),
        in_specs=[
            pl.BlockSpec((_EB, ED), lambda i: (i, 0)),
            pl.BlockSpec((ED, 2), lambda i: (0, 0)),
            pl.BlockSpec((1, 2), lambda i: (0, 0)),
        ],
        out_specs=[
            pl.BlockSpec((_EB, 1), lambda i: (i, 0)),
            pl.BlockSpec((_EB, 1), lambda i: (i, 0)),
        ],
        out_shape=[
            jax.ShapeDtypeStruct((E, 1), jnp.float32),
            jax.ShapeDtypeStruct((E, 1), jnp.float32),
        ],
    )(edge_attr, W12, b12)


def _spmm_body(hp_hbm, src_hbm, dst_hbm, w_hbm, out_hbm,
               sidx_v, didx_v, w_v, rows_v, buf_v, acc_sh, sem):
    cid = lax.axis_index("c")
    sid = lax.axis_index("s")
    wid = sid * NCORES + cid

    def _zb(r, carry):
        for c in range(DP // 16):
            buf_v[r, pl.ds(c * 16, 16)] = jnp.zeros((16,), jnp.float32)
        return carry

    lax.fori_loop(0, CPB, _zb, 0)
    for b in range(NCOPY):
        pltpu.sync_copy(buf_v, acc_sh.at[pl.ds(sid * ROWS_PER_SUB + b * CPB, CPB)])
    plsc.subcore_barrier()

    base_chunks = NCHUNKS // NW
    extra = NCHUNKS - base_chunks * NW
    my_n = base_chunks + jnp.where(wid < extra, 1, 0)

    def _chunk(t, carry):
        base = (wid + t * NW) * K
        pltpu.sync_copy(src_hbm.at[pl.ds(base, K)], sidx_v)
        pltpu.sync_copy(dst_hbm.at[pl.ds(base, K)], didx_v)
        pltpu.sync_copy(w_hbm.at[pl.ds(base, K)], w_v)
        pltpu.async_copy(hp_hbm.at[sidx_v], rows_v, sem).wait()

        def _scale(jj, c2):
            w16 = w_v[pl.ds(jj * 16, 16)]
            for l in range(16):
                wspl = jnp.full((16,), w16[l], jnp.float32)
                j = jj * 16 + l
                for c in range(DP // 16):
                    rows_v[j, pl.ds(c * 16, 16)] = rows_v[j, pl.ds(c * 16, 16)] * wspl
            return c2

        lax.fori_loop(0, K // 16, _scale, 0)
        pltpu.sync_copy(rows_v, acc_sh.at[didx_v], add=True)
        return carry

    lax.fori_loop(0, my_n, _chunk, 0)
    plsc.subcore_barrier()

    for b in range(NCOPY):
        r0 = sid * ROWS_PER_SUB + b * CPB
        pltpu.sync_copy(acc_sh.at[pl.ds(r0, CPB)], buf_v)
        pltpu.sync_copy(buf_v, out_hbm.at[cid, pl.ds(r0, CPB)])


_spmm = pl.kernel(
    _spmm_body,
    out_type=jax.ShapeDtypeStruct((NCORES, NP, DP), jnp.float32),
    mesh=plsc.VectorSubcoreMesh(core_axis_name="c", subcore_axis_name="s",
                                num_cores=NCORES, num_subcores=NSUB),
    scratch_types=[
        pltpu.VMEM((K,), jnp.int32),
        pltpu.VMEM((K,), jnp.int32),
        pltpu.VMEM((K,), jnp.float32),
        pltpu.VMEM((K, DP), jnp.float32),
        pltpu.VMEM((CPB, DP), jnp.float32),
        pltpu.VMEM_SHARED((NP, DP), jnp.float32),
        pltpu.SemaphoreType.DMA,
    ],
    compiler_params=pltpu.CompilerParams(use_tc_tiling_on_sc=False),
)


_RB = 512


def _layer_body(final, p_ref, hp_ref, wl_ref, bl_ref, g_ref, be_ref,
                wo_ref, bo_ref, out_ref):
    s = p_ref[0] + p_ref[1]
    c = s[:, D:D + 1]
    h = hp_ref[:, :D]
    agg = s[:, :D] - c * h
    z = jnp.dot(agg, wl_ref[...], preferred_element_type=jnp.float32) + bl_ref[...]
    r = jnp.maximum(z, 0.0)
    mu = jnp.mean(r, axis=-1, keepdims=True)
    dev = r - mu
    var = jnp.mean(dev * dev, axis=-1, keepdims=True)
    ln = dev * lax.rsqrt(var + 1e-5) * g_ref[...] + be_ref[...]
    h2 = ln + h
    if final:
        out_ref[...] = (
            jnp.dot(h2, wo_ref[...], preferred_element_type=jnp.float32) + bo_ref[...]
        )
    else:
        out_ref[:, :D] = h2
        out_ref[:, D:D + 1] = jnp.ones((_RB, 1), jnp.float32)
        out_ref[:, D + 1:] = jnp.zeros((_RB, DP - D - 1), jnp.float32)


def _layer_tc(final, P, hp, W_l, b_l, g, be, W_o, b_o):
    odim = D if final else DP
    return pl.pallas_call(
        functools.partial(_layer_body, final),
        grid=(NP // _RB,),
        in_specs=[
            pl.BlockSpec((NCORES, _RB, DP), lambda i: (0, i, 0)),
            pl.BlockSpec((_RB, DP), lambda i: (i, 0)),
            pl.BlockSpec((D, D), lambda i: (0, 0)),
            pl.BlockSpec((1, D), lambda i: (0, 0)),
            pl.BlockSpec((1, D), lambda i: (0, 0)),
            pl.BlockSpec((1, D), lambda i: (0, 0)),
            pl.BlockSpec((D, D), lambda i: (0, 0)),
            pl.BlockSpec((1, D), lambda i: (0, 0)),
        ],
        out_specs=pl.BlockSpec((_RB, odim), lambda i: (i, 0)),
        out_shape=jax.ShapeDtypeStruct((NP, odim), jnp.float32),
    )(P, hp, W_l, b_l, g, be, W_o, b_o)


def kernel(x, edge_index, edge_attr, W_e1, b_e1, W_l1, b_l1, g1, be1,
           W_e2, b_e2, W_l2, b_l2, g2, be2, W_out, b_out):
    src = edge_index[0]
    dst = edge_index[1]

    W12 = jnp.concatenate([W_e1, W_e2], axis=1)
    b12 = jnp.stack([b_e1[0], b_e2[0]]).reshape(1, 2)
    w1, w2 = _edge_weights(edge_attr, W12, b12)
    w1 = w1.reshape(E)
    w2 = w2.reshape(E)

    xp = jnp.zeros((NP, DP), jnp.float32)
    xp = xp.at[:N, :D].set(x)
    xp = xp.at[:, D].set(1.0)

    b_l1r = b_l1.reshape(1, D)
    g1r = g1.reshape(1, D)
    be1r = be1.reshape(1, D)
    b_l2r = b_l2.reshape(1, D)
    g2r = g2.reshape(1, D)
    be2r = be2.reshape(1, D)
    b_or = b_out.reshape(1, D)

    P1 = _spmm(xp, src, dst, w1)
    h1p = _layer_tc(False, P1, xp, W_l1, b_l1r, g1r, be1r, W_out, b_or)
    P2 = _spmm(h1p, src, dst, w2)
    out = _layer_tc(True, P2, h1p, W_l2, b_l2r, g2r, be2r, W_out, b_or)
    return out[:N]


# speedup vs baseline: 6.2533x; 1.5453x over previous
import functools

import jax
import jax.numpy as jnp
from jax import lax
from jax.experimental import pallas as pl
from jax.experimental.pallas import tpu as pltpu
from jax.experimental.pallas import tpu_sc as plsc

N = 10000
E = 320000
D = 128
ED = 16

NP = 10240
K = 80
NCORES = 2
NSUB = 16
NW = NCORES * NSUB
EPW = E // NW
NCH = EPW // K
IB = 2000
NBLK = EPW // IB
NCHB = IB // K
ROWS_PER_SUB = NP // NSUB
CPB = K
NCOPY = ROWS_PER_SUB // CPB
CN = NP // D


_EB = 512


def _edge_w_body(ea_ref, w12_ref, b12_ref, w1_ref, w2_ref):
    z = jnp.dot(w12_ref[...], ea_ref[...], preferred_element_type=jnp.float32)
    z = z + b12_ref[...]
    w = jnp.maximum(z, 0.0) + jnp.log(1.0 + jnp.exp(-jnp.abs(z)))
    w1_ref[...] = w[0]
    w2_ref[...] = w[1]


def _edge_weights(ea_t, W12t, b12):
    return pl.pallas_call(
        _edge_w_body,
        grid=(E // _EB,),
        in_specs=[
            pl.BlockSpec((ED, _EB), lambda i: (0, i)),
            pl.BlockSpec((2, ED), lambda i: (0, 0)),
            pl.BlockSpec((2, 1), lambda i: (0, 0)),
        ],
        out_specs=[
            pl.BlockSpec((_EB,), lambda i: (i,)),
            pl.BlockSpec((_EB,), lambda i: (i,)),
        ],
        out_shape=[
            jax.ShapeDtypeStruct((E,), jnp.float32),
            jax.ShapeDtypeStruct((E,), jnp.float32),
        ],
    )(ea_t, W12t, b12)


def _spmm_body(hp_hbm, src_hbm, dst2d_hbm, w_hbm, out_hbm, outc_hbm,
               sidx_v, didx_v, w_v, rows_a, rows_b, c_local, iota_v, acc_sh,
               c_sh, sem_a, sem_b):
    cid = lax.axis_index("c")
    sid = lax.axis_index("s")
    wid = sid * NCORES + cid
    e0 = wid * EPW

    def _zc(r, carry):
        for c in range(D // 16):
            c_local[r, pl.ds(c * 16, 16)] = jnp.zeros((16,), jnp.float32)
            rows_a[r, pl.ds(c * 16, 16)] = jnp.zeros((16,), jnp.float32)
        return carry

    lax.fori_loop(0, CN, _zc, 0)
    for r in range(CN // 16):
        iota_v[pl.ds(r * 16, 16)] = lax.iota(jnp.int32, 16) + (r * 16)
    for b in range(NCOPY):
        pltpu.sync_copy(rows_a, acc_sh.at[pl.ds(sid * ROWS_PER_SUB + b * CPB, CPB)])

    @pl.when(sid == 0)
    def _():
        pltpu.sync_copy(rows_a, c_sh)

    plsc.subcore_barrier()

    def _gather(t, rows, sem):
        pltpu.async_copy(hp_hbm.at[sidx_v.at[pl.ds(t * K, K)]], rows, sem)

    def _wait(rows, sem):
        pltpu.make_async_copy(hp_hbm.at[pl.ds(0, K)], rows, sem).wait()

    def _scale_scatter(t, rows):
        for jj in range(K // 16):
            w16 = w_v[pl.ds(t * K + jj * 16, 16)]
            didx16 = didx_v[t, pl.ds(jj * 16, 16)]
            plsc.addupdate_scatter(
                c_local,
                [lax.shift_right_logical(didx16, 7),
                 lax.bitwise_and(didx16, 127)],
                w16)
            for l in range(16):
                wspl = jnp.full((16,), w16[l], jnp.float32)
                j = jj * 16 + l
                for c in range(D // 16):
                    rows[j, pl.ds(c * 16, 16)] = rows[j, pl.ds(c * 16, 16)] * wspl
        pltpu.sync_copy(rows, acc_sh.at[didx_v.at[t]], add=True)

    def _block(blk, carry):
        eb = e0 + blk * IB
        pltpu.sync_copy(src_hbm.at[pl.ds(eb, IB)], sidx_v)
        pltpu.sync_copy(w_hbm.at[pl.ds(eb, IB)], w_v)
        pltpu.sync_copy(dst2d_hbm.at[wid * NBLK + blk], didx_v)
        _gather(0, rows_a, sem_a)

        def _pair(i, c2):
            t = i * 2
            _gather(t + 1, rows_b, sem_b)
            _wait(rows_a, sem_a)
            _scale_scatter(t, rows_a)
            _gather(t + 2, rows_a, sem_a)
            _wait(rows_b, sem_b)
            _scale_scatter(t + 1, rows_b)
            return c2

        lax.fori_loop(0, (NCHB - 1) // 2, _pair, 0)
        _wait(rows_a, sem_a)
        _scale_scatter(NCHB - 1, rows_a)
        return carry

    lax.fori_loop(0, NBLK, _block, 0)

    pltpu.sync_copy(c_local, c_sh.at[iota_v], add=True)
    plsc.subcore_barrier()

    for b in range(NCOPY):
        r0 = sid * ROWS_PER_SUB + b * CPB
        pltpu.sync_copy(acc_sh.at[pl.ds(r0, CPB)], rows_a)
        pltpu.sync_copy(rows_a, out_hbm.at[cid, pl.ds(r0, CPB)])
    @pl.when(sid == 0)
    def _():
        pltpu.sync_copy(c_sh, c_local)
        pltpu.sync_copy(c_local, outc_hbm.at[pl.ds(cid * CN, CN)])


_spmm = pl.kernel(
    _spmm_body,
    out_type=[
        jax.ShapeDtypeStruct((NCORES, NP, D), jnp.float32),
        jax.ShapeDtypeStruct((NCORES * CN, D), jnp.float32),
    ],
    mesh=plsc.VectorSubcoreMesh(core_axis_name="c", subcore_axis_name="s",
                                num_cores=NCORES, num_subcores=NSUB),
    scratch_types=[
        pltpu.VMEM((IB,), jnp.int32),
        pltpu.VMEM((NCHB, K), jnp.int32),
        pltpu.VMEM((IB,), jnp.float32),
        pltpu.VMEM((K, D), jnp.float32),
        pltpu.VMEM((K, D), jnp.float32),
        pltpu.VMEM((CN, D), jnp.float32),
        pltpu.VMEM((CN,), jnp.int32),
        pltpu.VMEM_SHARED((NP, D), jnp.float32),
        pltpu.VMEM_SHARED((CN, D), jnp.float32),
        pltpu.SemaphoreType.DMA,
        pltpu.SemaphoreType.DMA,
    ],
    compiler_params=pltpu.CompilerParams(needs_layout_passes=False),
)


_RB = 512


def _layer_body(final, p_ref, cd_ref, hp_ref, wl_ref, bl_ref, g_ref, be_ref,
                wo_ref, bo_ref, out_ref):
    h = hp_ref[...]
    agg = p_ref[0] + p_ref[1] - cd_ref[...]
    z = jnp.dot(agg, wl_ref[...], preferred_element_type=jnp.float32) + bl_ref[...]
    r = jnp.maximum(z, 0.0)
    mu = jnp.mean(r, axis=-1, keepdims=True)
    dev = r - mu
    var = jnp.mean(dev * dev, axis=-1, keepdims=True)
    ln = dev * lax.rsqrt(var + 1e-5) * g_ref[...] + be_ref[...]
    h2 = ln + h
    if final:
        out_ref[...] = (
            jnp.dot(h2, wo_ref[...], preferred_element_type=jnp.float32) + bo_ref[...]
        )
    else:
        out_ref[...] = h2


def _layer_tc(final, P, cd, hp, W_l, b_l, g, be, W_o, b_o):
    return pl.pallas_call(
        functools.partial(_layer_body, final),
        grid=(NP // _RB,),
        in_specs=[
            pl.BlockSpec((NCORES, _RB, D), lambda i: (0, i, 0)),
            pl.BlockSpec((_RB, D), lambda i: (i, 0)),
            pl.BlockSpec((_RB, D), lambda i: (i, 0)),
            pl.BlockSpec((D, D), lambda i: (0, 0)),
            pl.BlockSpec((1, D), lambda i: (0, 0)),
            pl.BlockSpec((1, D), lambda i: (0, 0)),
            pl.BlockSpec((1, D), lambda i: (0, 0)),
            pl.BlockSpec((D, D), lambda i: (0, 0)),
            pl.BlockSpec((1, D), lambda i: (0, 0)),
        ],
        out_specs=pl.BlockSpec((_RB, D), lambda i: (i, 0)),
        out_shape=jax.ShapeDtypeStruct((NP, D), jnp.float32),
    )(P, cd, hp, W_l, b_l, g, be, W_o, b_o)


def kernel(x, edge_index, edge_attr, W_e1, b_e1, W_l1, b_l1, g1, be1,
           W_e2, b_e2, W_l2, b_l2, g2, be2, W_out, b_out):
    src = edge_index[0]
    dst2d = edge_index[1].reshape(NW * NBLK, NCHB, K)

    W12t = jnp.stack([W_e1[:, 0], W_e2[:, 0]])
    b12 = jnp.stack([b_e1[0], b_e2[0]]).reshape(2, 1)
    w1, w2 = _edge_weights(edge_attr.T, W12t, b12)

    xp = jnp.zeros((NP, D), jnp.float32).at[:N, :].set(x)

    b_l1r = b_l1.reshape(1, D)
    g1r = g1.reshape(1, D)
    be1r = be1.reshape(1, D)
    b_l2r = b_l2.reshape(1, D)
    g2r = g2.reshape(1, D)
    be2r = be2.reshape(1, D)
    b_or = b_out.reshape(1, D)

    P1, C1 = _spmm(xp, src, dst2d, w1)
    cd1 = (C1[:CN] + C1[CN:]).reshape(NP, 1) * xp
    h1 = _layer_tc(False, P1, cd1, xp, W_l1, b_l1r, g1r, be1r, W_out, b_or)
    P2, C2 = _spmm(h1, src, dst2d, w2)
    cd2 = (C2[:CN] + C2[CN:]).reshape(NP, 1) * h1
    out = _layer_tc(True, P2, cd2, h1, W_l2, b_l2r, g2r, be2r, W_out, b_or)
    return out[:N]


# speedup vs baseline: 10.0891x; 1.6134x over previous
import functools

import jax
import jax.numpy as jnp
from jax import lax
from jax.experimental import pallas as pl
from jax.experimental.pallas import tpu as pltpu
from jax.experimental.pallas import tpu_sc as plsc

N = 10000
E = 320000
D = 128
ED = 16

NP = 10240
K = 80
NCORES = 2
NSUB = 16
NW = NCORES * NSUB
EPW = E // NW
NCH = EPW // K
IB = 2000
NBLK = EPW // IB
NCHB = IB // K
ROWS_PER_SUB = NP // NSUB
CPB = K
NCOPY = ROWS_PER_SUB // CPB
CN = NP // D


_EB = 12800


def _edge_w_body(ea_ref, w12_ref, b12_ref, w_ref):
    z = jnp.dot(w12_ref[...], ea_ref[...], preferred_element_type=jnp.float32)
    z = z + b12_ref[...]
    w_ref[...] = jnp.maximum(z, 0.0) + jnp.log(1.0 + jnp.exp(-jnp.abs(z)))


def _edge_weights(ea_t, W12t, b12):
    return pl.pallas_call(
        _edge_w_body,
        grid=(E // _EB,),
        in_specs=[
            pl.BlockSpec((ED, _EB), lambda i: (0, i)),
            pl.BlockSpec((2, ED), lambda i: (0, 0)),
            pl.BlockSpec((2, 1), lambda i: (0, 0)),
        ],
        out_specs=pl.BlockSpec((2, _EB), lambda i: (0, i)),
        out_shape=jax.ShapeDtypeStruct((2, E), jnp.float32),
    )(ea_t, W12t, b12)


def _spmm_body(hp_hbm, src_hbm, dst2d_hbm, w_hbm, out_hbm, outc_hbm,
               sidx_v, didx_v, w_v, rows_a, rows_b, c_local, iota_v, acc_sh,
               c_sh, sem_a, sem_b):
    cid = lax.axis_index("c")
    sid = lax.axis_index("s")
    wid = sid * NCORES + cid
    e0 = wid * EPW

    def _zc(r, carry):
        for c in range(D // 16):
            c_local[r, pl.ds(c * 16, 16)] = jnp.zeros((16,), jnp.float32)
            rows_a[r, pl.ds(c * 16, 16)] = jnp.zeros((16,), jnp.float32)
        return carry

    lax.fori_loop(0, CN, _zc, 0)
    for r in range(CN // 16):
        iota_v[pl.ds(r * 16, 16)] = lax.iota(jnp.int32, 16) + (r * 16)
    for b in range(NCOPY):
        pltpu.sync_copy(rows_a, acc_sh.at[pl.ds(sid * ROWS_PER_SUB + b * CPB, CPB)])

    @pl.when(sid == 0)
    def _():
        pltpu.sync_copy(rows_a, c_sh)

    plsc.subcore_barrier()

    def _gather(t, rows, sem):
        pltpu.async_copy(hp_hbm.at[sidx_v.at[pl.ds(t * K, K)]], rows, sem)

    def _wait(rows, sem):
        pltpu.make_async_copy(hp_hbm.at[pl.ds(0, K)], rows, sem).wait()

    def _scale_scatter(t, rows):
        for jj in range(K // 16):
            w16 = w_v[pl.ds(t * K + jj * 16, 16)]
            didx16 = didx_v[t, pl.ds(jj * 16, 16)]
            plsc.addupdate_scatter(
                c_local,
                [lax.shift_right_logical(didx16, 7),
                 lax.bitwise_and(didx16, 127)],
                w16)
            for l in range(16):
                wspl = jnp.full((16,), w16[l], jnp.float32)
                j = jj * 16 + l
                for c in range(D // 16):
                    rows[j, pl.ds(c * 16, 16)] = rows[j, pl.ds(c * 16, 16)] * wspl
        pltpu.sync_copy(rows, acc_sh.at[didx_v.at[t]], add=True)

    def _block(blk, carry):
        eb = e0 + blk * IB
        pltpu.sync_copy(src_hbm.at[pl.ds(eb, IB)], sidx_v)
        pltpu.sync_copy(w_hbm.at[pl.ds(eb, IB)], w_v)
        pltpu.sync_copy(dst2d_hbm.at[wid * NBLK + blk], didx_v)
        _gather(0, rows_a, sem_a)

        def _pair(i, c2):
            t = i * 2
            _gather(t + 1, rows_b, sem_b)
            _wait(rows_a, sem_a)
            _scale_scatter(t, rows_a)
            _gather(t + 2, rows_a, sem_a)
            _wait(rows_b, sem_b)
            _scale_scatter(t + 1, rows_b)
            return c2

        lax.fori_loop(0, (NCHB - 1) // 2, _pair, 0)
        _wait(rows_a, sem_a)
        _scale_scatter(NCHB - 1, rows_a)
        return carry

    lax.fori_loop(0, NBLK, _block, 0)

    pltpu.sync_copy(c_local, c_sh.at[iota_v], add=True)
    plsc.subcore_barrier()

    for b in range(NCOPY):
        r0 = sid * ROWS_PER_SUB + b * CPB
        pltpu.sync_copy(acc_sh.at[pl.ds(r0, CPB)], rows_a)
        pltpu.sync_copy(rows_a, out_hbm.at[cid, pl.ds(r0, CPB)])
    @pl.when(sid == 0)
    def _():
        pltpu.sync_copy(c_sh, c_local)
        pltpu.sync_copy(c_local, outc_hbm.at[pl.ds(cid * CN, CN)])


_spmm = pl.kernel(
    _spmm_body,
    out_type=[
        jax.ShapeDtypeStruct((NCORES, NP, D), jnp.float32),
        jax.ShapeDtypeStruct((NCORES * CN, D), jnp.float32),
    ],
    mesh=plsc.VectorSubcoreMesh(core_axis_name="c", subcore_axis_name="s",
                                num_cores=NCORES, num_subcores=NSUB),
    scratch_types=[
        pltpu.VMEM((IB,), jnp.int32),
        pltpu.VMEM((NCHB, K), jnp.int32),
        pltpu.VMEM((IB,), jnp.float32),
        pltpu.VMEM((K, D), jnp.float32),
        pltpu.VMEM((K, D), jnp.float32),
        pltpu.VMEM((CN, D), jnp.float32),
        pltpu.VMEM((CN,), jnp.int32),
        pltpu.VMEM_SHARED((NP, D), jnp.float32),
        pltpu.VMEM_SHARED((CN, D), jnp.float32),
        pltpu.SemaphoreType.DMA,
        pltpu.SemaphoreType.DMA,
    ],
    compiler_params=pltpu.CompilerParams(needs_layout_passes=False),
)


_RB = 512


def _layer_body(final, p_ref, cd_ref, hp_ref, wl_ref, bl_ref, g_ref, be_ref,
                wo_ref, bo_ref, out_ref):
    h = hp_ref[...]
    agg = p_ref[0] + p_ref[1] - cd_ref[...]
    z = jnp.dot(agg, wl_ref[...], preferred_element_type=jnp.float32) + bl_ref[...]
    r = jnp.maximum(z, 0.0)
    mu = jnp.mean(r, axis=-1, keepdims=True)
    dev = r - mu
    var = jnp.mean(dev * dev, axis=-1, keepdims=True)
    ln = dev * lax.rsqrt(var + 1e-5) * g_ref[...] + be_ref[...]
    h2 = ln + h
    if final:
        out_ref[...] = (
            jnp.dot(h2, wo_ref[...], preferred_element_type=jnp.float32) + bo_ref[...]
        )
    else:
        out_ref[...] = h2


def _layer_tc(final, P, cd, hp, W_l, b_l, g, be, W_o, b_o):
    return pl.pallas_call(
        functools.partial(_layer_body, final),
        grid=(NP // _RB,),
        in_specs=[
            pl.BlockSpec((NCORES, _RB, D), lambda i: (0, i, 0)),
            pl.BlockSpec((_RB, D), lambda i: (i, 0)),
            pl.BlockSpec((_RB, D), lambda i: (i, 0)),
            pl.BlockSpec((D, D), lambda i: (0, 0)),
            pl.BlockSpec((1, D), lambda i: (0, 0)),
            pl.BlockSpec((1, D), lambda i: (0, 0)),
            pl.BlockSpec((1, D), lambda i: (0, 0)),
            pl.BlockSpec((D, D), lambda i: (0, 0)),
            pl.BlockSpec((1, D), lambda i: (0, 0)),
        ],
        out_specs=pl.BlockSpec((_RB, D), lambda i: (i, 0)),
        out_shape=jax.ShapeDtypeStruct((NP, D), jnp.float32),
    )(P, cd, hp, W_l, b_l, g, be, W_o, b_o)


def kernel(x, edge_index, edge_attr, W_e1, b_e1, W_l1, b_l1, g1, be1,
           W_e2, b_e2, W_l2, b_l2, g2, be2, W_out, b_out):
    src = edge_index[0]
    dst2d = edge_index[1].reshape(NW * NBLK, NCHB, K)

    W12t = jnp.stack([W_e1[:, 0], W_e2[:, 0]])
    b12 = jnp.stack([b_e1[0], b_e2[0]]).reshape(2, 1)
    w12 = _edge_weights(edge_attr.T, W12t, b12)
    w1 = w12[0]
    w2 = w12[1]

    xp = jnp.zeros((NP, D), jnp.float32).at[:N, :].set(x)

    b_l1r = b_l1.reshape(1, D)
    g1r = g1.reshape(1, D)
    be1r = be1.reshape(1, D)
    b_l2r = b_l2.reshape(1, D)
    g2r = g2.reshape(1, D)
    be2r = be2.reshape(1, D)
    b_or = b_out.reshape(1, D)

    P1, C1 = _spmm(xp, src, dst2d, w1)
    cd1 = (C1[:CN] + C1[CN:]).reshape(NP, 1) * xp
    h1 = _layer_tc(False, P1, cd1, xp, W_l1, b_l1r, g1r, be1r, W_out, b_or)
    P2, C2 = _spmm(h1, src, dst2d, w2)
    cd2 = (C2[:CN] + C2[CN:]).reshape(NP, 1) * h1
    out = _layer_tc(True, P2, cd2, h1, W_l2, b_l2r, g2r, be2r, W_out, b_or)
    return out[:N]
